# baseline (speedup 1.0000x reference)
import functools

import jax
import jax.numpy as jnp
from jax import lax
from jax.experimental import pallas as pl
from jax.experimental.pallas import tpu as pltpu
from jax.experimental.pallas import tpu_sc as plsc

V = 100000
D = 300
DP = 384
B = 1024
NSL = 19
TPAD = 24
ND = 200
RB = 1000


def _tail_body(t_ref, o_ref):
    o_ref[...] = jnp.concatenate(
        [t_ref[:, pl.ds(0, D - 256)],
         jnp.zeros((RB, 128 - (D - 256)), jnp.float32)], axis=1)


def _tail_table(tbl):
    return pl.pallas_call(
        _tail_body,
        grid=(V // RB,),
        in_specs=[pl.BlockSpec((RB, 128), lambda i: (i, 2))],
        out_specs=pl.BlockSpec((RB, 128), lambda i: (i, 0)),
        out_shape=jax.ShapeDtypeStruct((V, 128), jnp.float32),
    )(tbl)


def _accum_rows(buf, nrows, accs):
    def body(r, a):
        return tuple(a[i] + buf[r, pl.ds(i * 16, 16)] for i in range(NSL))
    return lax.fori_loop(0, nrows, body, accs)


def _zero_accs():
    return tuple(jnp.zeros((16,), jnp.float32) for _ in range(NSL))


def _make_sc_pool(stride, chunks):
    info = plsc.get_sparse_core_info()
    nc, ns = info.num_cores, info.num_subcores
    nw = nc * ns
    bw = B // nw
    ngrp = bw // 8
    cmax = max(c[1] for c in chunks)
    nst = len(chunks)

    mesh = plsc.VectorSubcoreMesh(core_axis_name="c", subcore_axis_name="s")

    @functools.partial(
        pl.kernel,
        mesh=mesh,
        compiler_params=pltpu.CompilerParams(use_tc_tiling_on_sc=True,
                                             needs_layout_passes=False),
        out_type=jax.ShapeDtypeStruct((B, DP), jnp.float32),
        scratch_types=[
            pltpu.VMEM((B // nw * stride,), jnp.int32),
            pltpu.VMEM((cmax, DP), jnp.float32),
            pltpu.VMEM((cmax, DP), jnp.float32),
            pltpu.VMEM((8, DP), jnp.float32),
            pltpu.SemaphoreType.DMA,
            pltpu.SemaphoreType.DMA,
            pltpu.SemaphoreType.DMA,
        ],
    )
    def sc_pool(idx, tbl, tail, out,
                iv, buf_a, buf_b, ostage, sem_a, sem_b, sem_f):
        wid = lax.axis_index("s") * nc + lax.axis_index("c")
        base = wid * bw

        pltpu.sync_copy(idx.at[pl.ds(base * stride, bw * stride)], iv)

        for jm in range(8):
            for k in range(5):
                ostage[jm, pl.ds(304 + k * 16, 16)] = (
                    jnp.zeros((16,), jnp.float32))

        bufs = (buf_a, buf_b)
        sems = (sem_a, sem_b)

        def gather_cps(s, col, b):
            off, n, _ = chunks[s]
            ixs = iv.at[pl.ds(col * stride + off, n)]
            return (
                pltpu.make_async_copy(
                    tbl.at[ixs, pl.ds(0, 256)],
                    bufs[b].at[pl.ds(0, n), pl.ds(0, 256)], sems[b]),
                pltpu.make_async_copy(
                    tail.at[ixs],
                    bufs[b].at[pl.ds(0, n), pl.ds(256, 128)], sems[b]))

        def gather_start(s, col, b):
            for cp in gather_cps(s, col, b):
                cp.start()

        def gather_wait(s, col, b):
            for cp in gather_cps(s, col, b):
                cp.wait()

        def flush_cp(g):
            return pltpu.make_async_copy(
                ostage, out.at[pl.ds(base + g * 8, 8)], sem_f)

        flush_cp(0).start()
        gather_start(0, 0, 0)

        def grp_body(g, carry):
            flush_cp(g).wait()

            unroll = 2 if nst % 2 else 1

            def col_body(ji, carry2):
                for u in range(unroll):
                    jm = ji * unroll + u
                    j = g * 8 + jm
                    accs = _zero_accs()
                    for s, (off, n, na) in enumerate(chunks):
                        p = (u * nst + s) % 2
                        nxt = (s + 1) % nst
                        last_of_iter = (u == unroll - 1) and (s == nst - 1)
                        ncol = (jnp.minimum(j + 1, bw - 1)
                                if last_of_iter else
                                (j if s + 1 < nst else j + 1))
                        gather_start(nxt, ncol, (p + 1) % 2)
                        gather_wait(s, j, p)
                        accs = _accum_rows(bufs[p], na, accs)
                    for i in range(NSL):
                        ostage[jm, pl.ds(i * 16, 16)] = accs[i]
                return carry2

            lax.fori_loop(0, 8 // unroll, col_body, 0)
            flush_cp(g).start()
            return carry

        lax.fori_loop(0, ngrp, grp_body, 0)

        gather_wait(0, bw - 1, 0)
        flush_cp(ngrp - 1).wait()

    return sc_pool


def _mlp_body(jt, jd, ct, cr, p0, p1, p2, p3,
              w1, b1, w2, b2, w3, b3, out):
    h = jnp.broadcast_to(b1[...], (B, 400))
    zpad = jnp.zeros((DP - D, 400), jnp.float32)
    for t, (idx, pooled) in enumerate(
            zip((jt, jd, ct, cr), (p0, p1, p2, p3))):
        cnt = jnp.sum((idx[...] != 1).astype(jnp.float32), axis=0)
        x = pooled[...] / cnt[:, None]
        w1t = jnp.concatenate([w1[pl.ds(t * D, D), :], zpad], axis=0)
        h = h + jnp.dot(x, w1t, preferred_element_type=jnp.float32)
    h = jax.nn.relu(h)
    h = jax.nn.relu(jnp.dot(h, w2[...], preferred_element_type=jnp.float32)
                    + b2[...])
    h = jax.nn.relu(jnp.dot(h, w3[...], preferred_element_type=jnp.float32)
                    + b3[...])
    out[...] = jax.nn.sigmoid(h)


def _flatten_idx(idx, npad):
    n = idx.shape[0]
    cols = idx.T
    if npad > n:
        k = jnp.arange(npad - n, dtype=jnp.int32)[None, :]
        c = jnp.arange(B, dtype=jnp.int32)[:, None]
        fill = (c * 4 + k) % V
        cols = jnp.concatenate([cols, fill], axis=1)
    return cols.reshape(-1)


_TITLE_CHUNKS = ((0, TPAD, 20),)
_DESC_CHUNKS = ((0, 104, 104), (104, 96, 96))


def kernel(job_title, job_description, candidate_title, candidate_resume,
           emb_job_title, emb_job_description, emb_candidate_title,
           emb_candidate_resume, W1, b1, W2, b2, W3, b3):
    jt = job_title.astype(jnp.int32)
    jd = job_description.astype(jnp.int32)
    ct = candidate_title.astype(jnp.int32)
    cr = candidate_resume.astype(jnp.int32)

    sc_title = _make_sc_pool(TPAD, _TITLE_CHUNKS)
    sc_desc = _make_sc_pool(ND, _DESC_CHUNKS)

    pooled = {}
    for key, idx, npad, tbl, sc in (
            ("jd", jd, ND, emb_job_description, sc_desc),
            ("cr", cr, ND, emb_candidate_resume, sc_desc),
            ("jt", jt, TPAD, emb_job_title, sc_title),
            ("ct", ct, TPAD, emb_candidate_title, sc_title)):
        flat = _flatten_idx(idx, npad)
        tail = _tail_table(tbl)
        pooled[key] = sc(flat, tbl, tail)
    pooled = [pooled["jt"], pooled["jd"], pooled["ct"], pooled["cr"]]

    out = pl.pallas_call(
        _mlp_body,
        out_shape=jax.ShapeDtypeStruct((B, 1), jnp.float32),
    )(jt, jd, ct, cr, *pooled, W1,
      b1.reshape(1, 400), W2, b2.reshape(1, 100), W3, b3.reshape(1, 1))
    return out


# speedup vs baseline: 1.0010x; 1.0010x over previous
import functools

import jax
import jax.numpy as jnp
from jax import lax
from jax.experimental import pallas as pl
from jax.experimental.pallas import tpu as pltpu
from jax.experimental.pallas import tpu_sc as plsc

V = 100000
D = 300
DP = 384
B = 1024
NSL = 19
TPAD = 24
ND = 200
RB = 1000


def _tail_body(t_ref, o_ref):
    o_ref[...] = jnp.concatenate(
        [t_ref[:, pl.ds(0, D - 256)],
         jnp.zeros((RB, 128 - (D - 256)), jnp.float32)], axis=1)


def _tail_table(tbl):
    return pl.pallas_call(
        _tail_body,
        grid=(V // RB,),
        in_specs=[pl.BlockSpec((RB, 128), lambda i: (i, 2))],
        out_specs=pl.BlockSpec((RB, 128), lambda i: (i, 0)),
        out_shape=jax.ShapeDtypeStruct((V, 128), jnp.float32),
    )(tbl)


def _accum_rows(buf, nrows, accs):
    def body(r, a):
        return tuple(a[i] + buf[r, pl.ds(i * 16, 16)] for i in range(NSL))
    return lax.fori_loop(0, nrows, body, accs)


def _zero_accs():
    return tuple(jnp.zeros((16,), jnp.float32) for _ in range(NSL))


def _make_sc_pool(stride, chunks):
    info = plsc.get_sparse_core_info()
    nc, ns = info.num_cores, info.num_subcores
    nw = nc * ns
    bw = B // nw
    ngrp = bw // 8
    cmax = max(c[1] for c in chunks)
    nst = len(chunks)

    mesh = plsc.VectorSubcoreMesh(core_axis_name="c", subcore_axis_name="s")

    @functools.partial(
        pl.kernel,
        mesh=mesh,
        compiler_params=pltpu.CompilerParams(use_tc_tiling_on_sc=True,
                                             needs_layout_passes=True),
        out_type=jax.ShapeDtypeStruct((B, DP), jnp.float32),
        scratch_types=[
            pltpu.VMEM((B // nw * stride,), jnp.int32),
            pltpu.VMEM((cmax, DP), jnp.float32),
            pltpu.VMEM((cmax, DP), jnp.float32),
            pltpu.VMEM((8, DP), jnp.float32),
            pltpu.SemaphoreType.DMA,
            pltpu.SemaphoreType.DMA,
            pltpu.SemaphoreType.DMA,
        ],
    )
    def sc_pool(idx, tbl, tail, out,
                iv, buf_a, buf_b, ostage, sem_a, sem_b, sem_f):
        wid = lax.axis_index("s") * nc + lax.axis_index("c")
        base = wid * bw

        pltpu.sync_copy(idx.at[pl.ds(base * stride, bw * stride)], iv)

        for jm in range(8):
            for k in range(5):
                ostage[jm, pl.ds(304 + k * 16, 16)] = (
                    jnp.zeros((16,), jnp.float32))

        bufs = (buf_a, buf_b)
        sems = (sem_a, sem_b)

        def gather_cps(s, col, b):
            off, n, _ = chunks[s]
            ixs = iv.at[pl.ds(col * stride + off, n)]
            return (
                pltpu.make_async_copy(
                    tbl.at[ixs, pl.ds(0, 256)],
                    bufs[b].at[pl.ds(0, n), pl.ds(0, 256)], sems[b]),
                pltpu.make_async_copy(
                    tail.at[ixs],
                    bufs[b].at[pl.ds(0, n), pl.ds(256, 128)], sems[b]))

        def gather_start(s, col, b):
            for cp in gather_cps(s, col, b):
                cp.start()

        def gather_wait(s, col, b):
            for cp in gather_cps(s, col, b):
                cp.wait()

        def flush_cp(g):
            return pltpu.make_async_copy(
                ostage, out.at[pl.ds(base + g * 8, 8)], sem_f)

        flush_cp(0).start()
        gather_start(0, 0, 0)

        def grp_body(g, carry):
            flush_cp(g).wait()

            unroll = 2 if nst % 2 else 1

            def col_body(ji, carry2):
                for u in range(unroll):
                    jm = ji * unroll + u
                    j = g * 8 + jm
                    accs = _zero_accs()
                    for s, (off, n, na) in enumerate(chunks):
                        p = (u * nst + s) % 2
                        nxt = (s + 1) % nst
                        last_of_iter = (u == unroll - 1) and (s == nst - 1)
                        ncol = (jnp.minimum(j + 1, bw - 1)
                                if last_of_iter else
                                (j if s + 1 < nst else j + 1))
                        gather_start(nxt, ncol, (p + 1) % 2)
                        gather_wait(s, j, p)
                        accs = _accum_rows(bufs[p], na, accs)
                    for i in range(NSL):
                        ostage[jm, pl.ds(i * 16, 16)] = accs[i]
                return carry2

            lax.fori_loop(0, 8 // unroll, col_body, 0)
            flush_cp(g).start()
            return carry

        lax.fori_loop(0, ngrp, grp_body, 0)

        gather_wait(0, bw - 1, 0)
        flush_cp(ngrp - 1).wait()

    return sc_pool


def _mlp_body(jt, jd, ct, cr, p0, p1, p2, p3,
              w1, b1, w2, b2, w3, b3, out):
    h = jnp.broadcast_to(b1[...], (B, 400))
    zpad = jnp.zeros((DP - D, 400), jnp.float32)
    for t, (idx, pooled) in enumerate(
            zip((jt, jd, ct, cr), (p0, p1, p2, p3))):
        cnt = jnp.sum((idx[...] != 1).astype(jnp.float32), axis=0)
        x = pooled[...] / cnt[:, None]
        w1t = jnp.concatenate([w1[pl.ds(t * D, D), :], zpad], axis=0)
        h = h + jnp.dot(x, w1t, preferred_element_type=jnp.float32)
    h = jax.nn.relu(h)
    h = jax.nn.relu(jnp.dot(h, w2[...], preferred_element_type=jnp.float32)
                    + b2[...])
    h = jax.nn.relu(jnp.dot(h, w3[...], preferred_element_type=jnp.float32)
                    + b3[...])
    out[...] = jax.nn.sigmoid(h)


def _flatten_idx(idx, npad):
    n = idx.shape[0]
    cols = idx.T
    if npad > n:
        k = jnp.arange(npad - n, dtype=jnp.int32)[None, :]
        c = jnp.arange(B, dtype=jnp.int32)[:, None]
        fill = (c * 4 + k) % V
        cols = jnp.concatenate([cols, fill], axis=1)
    return cols.reshape(-1)


_TITLE_CHUNKS = ((0, TPAD, 20),)
_DESC_CHUNKS = ((0, 104, 104), (104, 96, 96))


def kernel(job_title, job_description, candidate_title, candidate_resume,
           emb_job_title, emb_job_description, emb_candidate_title,
           emb_candidate_resume, W1, b1, W2, b2, W3, b3):
    jt = job_title.astype(jnp.int32)
    jd = job_description.astype(jnp.int32)
    ct = candidate_title.astype(jnp.int32)
    cr = candidate_resume.astype(jnp.int32)

    sc_title = _make_sc_pool(TPAD, _TITLE_CHUNKS)
    sc_desc = _make_sc_pool(ND, _DESC_CHUNKS)

    pooled = {}
    for key, idx, npad, tbl, sc in (
            ("jd", jd, ND, emb_job_description, sc_desc),
            ("cr", cr, ND, emb_candidate_resume, sc_desc),
            ("jt", jt, TPAD, emb_job_title, sc_title),
            ("ct", ct, TPAD, emb_candidate_title, sc_title)):
        flat = _flatten_idx(idx, npad)
        tail = _tail_table(tbl)
        pooled[key] = sc(flat, tbl, tail)
    pooled = [pooled["jt"], pooled["jd"], pooled["ct"], pooled["cr"]]

    out = pl.pallas_call(
        _mlp_body,
        out_shape=jax.ShapeDtypeStruct((B, 1), jnp.float32),
    )(jt, jd, ct, cr, *pooled, W1,
      b1.reshape(1, 400), W2, b2.reshape(1, 100), W3, b3.reshape(1, 1))
    return out


# speedup vs baseline: 1.0221x; 1.0212x over previous
import functools

import jax
import jax.numpy as jnp
from jax import lax
from jax.experimental import pallas as pl
from jax.experimental.pallas import tpu as pltpu
from jax.experimental.pallas import tpu_sc as plsc

V = 100000
D = 300
DP = 384
B = 1024
NSL = 19
TPAD = 24
ND = 200
RB = 1000


def _tail_body(t_ref, o_ref):
    o_ref[...] = jnp.concatenate(
        [t_ref[:, pl.ds(0, D - 256)],
         jnp.zeros((RB, 128 - (D - 256)), jnp.float32)], axis=1)


def _tail_table(tbl):
    return pl.pallas_call(
        _tail_body,
        grid=(V // RB,),
        in_specs=[pl.BlockSpec((RB, 128), lambda i: (i, 2))],
        out_specs=pl.BlockSpec((RB, 128), lambda i: (i, 0)),
        out_shape=jax.ShapeDtypeStruct((V, 128), jnp.float32),
    )(tbl)


def _accum_rows(buf, nrows, accs):
    def body(r, a):
        return tuple(a[i] + buf[r, pl.ds(i * 16, 16)] for i in range(NSL))
    return lax.fori_loop(0, nrows, body, accs)


def _zero_accs():
    return tuple(jnp.zeros((16,), jnp.float32) for _ in range(NSL))


def _make_sc_pool(stride, chunks):
    info = plsc.get_sparse_core_info()
    nc, ns = info.num_cores, info.num_subcores
    nw = nc * ns
    bw = B // nw
    ngrp = bw // 8
    cmax = max(c[1] for c in chunks)
    nst = len(chunks)

    mesh = plsc.VectorSubcoreMesh(core_axis_name="c", subcore_axis_name="s")

    @functools.partial(
        pl.kernel,
        mesh=mesh,
        compiler_params=pltpu.CompilerParams(use_tc_tiling_on_sc=True,
                                             needs_layout_passes=True),
        out_type=jax.ShapeDtypeStruct((B, DP), jnp.float32),
        scratch_types=[
            pltpu.VMEM((B // nw * stride,), jnp.int32),
            pltpu.VMEM((cmax, DP), jnp.float32),
            pltpu.VMEM((cmax, DP), jnp.float32),
            pltpu.VMEM((8, DP), jnp.float32),
            pltpu.SemaphoreType.DMA,
            pltpu.SemaphoreType.DMA,
            pltpu.SemaphoreType.DMA,
        ],
    )
    def sc_pool(idx, tbl, tail, out,
                iv, buf_a, buf_b, ostage, sem_a, sem_b, sem_f):
        wid = lax.axis_index("s") * nc + lax.axis_index("c")
        base = wid * bw

        pltpu.sync_copy(idx.at[pl.ds(base * stride, bw * stride)], iv)

        for jm in range(8):
            for k in range(5):
                ostage[jm, pl.ds(304 + k * 16, 16)] = (
                    jnp.zeros((16,), jnp.float32))

        bufs = (buf_a, buf_b)
        sems = (sem_a, sem_b)

        def gather_cps(s, col, b):
            off, n, _ = chunks[s]
            ixs = iv.at[pl.ds(col * stride + off, n)]
            return (
                pltpu.make_async_copy(
                    tbl.at[ixs, pl.ds(0, 256)],
                    bufs[b].at[pl.ds(0, n), pl.ds(0, 256)], sems[b]),
                pltpu.make_async_copy(
                    tail.at[ixs],
                    bufs[b].at[pl.ds(0, n), pl.ds(256, 128)], sems[b]))

        def gather_start(s, col, b):
            for cp in gather_cps(s, col, b):
                cp.start()

        def gather_wait(s, col, b):
            for cp in gather_cps(s, col, b):
                cp.wait()

        def flush_cp(g):
            return pltpu.make_async_copy(
                ostage, out.at[pl.ds(base + g * 8, 8)], sem_f)

        flush_cp(0).start()
        gather_start(0, 0, 0)

        def grp_body(g, carry):
            flush_cp(g).wait()

            unroll = 2 if nst % 2 else 1

            def col_body(ji, carry2):
                for u in range(unroll):
                    jm = ji * unroll + u
                    j = g * 8 + jm
                    accs = _zero_accs()
                    for s, (off, n, na) in enumerate(chunks):
                        p = (u * nst + s) % 2
                        nxt = (s + 1) % nst
                        last_of_iter = (u == unroll - 1) and (s == nst - 1)
                        ncol = (jnp.minimum(j + 1, bw - 1)
                                if last_of_iter else
                                (j if s + 1 < nst else j + 1))
                        gather_start(nxt, ncol, (p + 1) % 2)
                        gather_wait(s, j, p)
                        accs = _accum_rows(bufs[p], na, accs)
                    for i in range(NSL):
                        ostage[jm, pl.ds(i * 16, 16)] = accs[i]
                return carry2

            lax.fori_loop(0, 8 // unroll, col_body, 0)
            flush_cp(g).start()
            return carry

        lax.fori_loop(0, ngrp, grp_body, 0)

        gather_wait(0, bw - 1, 0)
        flush_cp(ngrp - 1).wait()

    return sc_pool


def _mlp_body(jt, jd, ct, cr, p0, p1, p2, p3,
              w1, b1, w2, b2, w3, b3, out):
    h = jnp.broadcast_to(b1[...], (B, 400))
    zpad = jnp.zeros((DP - D, 400), jnp.float32)
    for t, (idx, pooled) in enumerate(
            zip((jt, jd, ct, cr), (p0, p1, p2, p3))):
        cnt = jnp.sum((idx[...] != 1).astype(jnp.float32), axis=0)
        x = pooled[...] / cnt[:, None]
        w1t = jnp.concatenate([w1[pl.ds(t * D, D), :], zpad], axis=0)
        h = h + jnp.dot(x, w1t, preferred_element_type=jnp.float32)
    h = jax.nn.relu(h)
    h = jax.nn.relu(jnp.dot(h, w2[...], preferred_element_type=jnp.float32)
                    + b2[...])
    h = jax.nn.relu(jnp.dot(h, w3[...], preferred_element_type=jnp.float32)
                    + b3[...])
    out[...] = jax.nn.sigmoid(h)


def _flatten_idx(idx, npad):
    n = idx.shape[0]
    cols = idx.T
    if npad > n:
        k = jnp.arange(npad - n, dtype=jnp.int32)[None, :]
        c = jnp.arange(B, dtype=jnp.int32)[:, None]
        fill = (c * 4 + k) % V
        cols = jnp.concatenate([cols, fill], axis=1)
    return cols.reshape(-1)


_TITLE_CHUNKS = ((0, TPAD, 20),)
_DESC_CHUNKS = ((0, 104, 104), (104, 96, 96))


def kernel(job_title, job_description, candidate_title, candidate_resume,
           emb_job_title, emb_job_description, emb_candidate_title,
           emb_candidate_resume, W1, b1, W2, b2, W3, b3):
    jt = job_title.astype(jnp.int32)
    jd = job_description.astype(jnp.int32)
    ct = candidate_title.astype(jnp.int32)
    cr = candidate_resume.astype(jnp.int32)

    sc_title = _make_sc_pool(TPAD, _TITLE_CHUNKS)
    sc_desc = _make_sc_pool(ND, _DESC_CHUNKS)

    pooled = {}
    for key, idx, npad, tbl, sc in (
            ("jd", jd, ND, emb_job_description, sc_desc),
            ("cr", cr, ND, emb_candidate_resume, sc_desc),
            ("jt", jt, TPAD, emb_job_title, sc_title),
            ("ct", ct, TPAD, emb_candidate_title, sc_title)):
        flat = _flatten_idx(idx, npad)
        tail = jnp.concatenate(
            [lax.slice(tbl, (0, 256), (V, D)),
             jnp.zeros((V, 128 - (D - 256)), jnp.float32)], axis=1)
        pooled[key] = sc(flat, tbl, tail)
    pooled = [pooled["jt"], pooled["jd"], pooled["ct"], pooled["cr"]]

    out = pl.pallas_call(
        _mlp_body,
        out_shape=jax.ShapeDtypeStruct((B, 1), jnp.float32),
    )(jt, jd, ct, cr, *pooled, W1,
      b1.reshape(1, 400), W2, b2.reshape(1, 100), W3, b3.reshape(1, 1))
    return out
